# baseline (speedup 1.0000x reference)
import functools

import jax
import jax.numpy as jnp
from jax import lax
from jax.experimental import pallas as pl
from jax.experimental.pallas import tpu as pltpu
from jax.experimental.pallas import tpu_sc as plsc

N = 100000
G = 512
DS = 128
VI = 16
DV = 3 * VI

NC, NS = 2, 16
NW = NC * NS

C = 128
VR = C * DV // DS
NB_FULL = N // C
TAIL = N - NB_FULL * C
T_STEPS = -(-NB_FULL // NW)
ZR = G // NS
GP = G + 8
NBUF = 3


def _segsum_body(xs_hbm, xv_hbm, i_hbm, ps_hbm, pv_hbm,
                 xs0, xs1, xs2, xv0, xv1, pad0, pad1,
                 idx2, idxt,
                 shared_s, shared_v,
                 gsem0, gsem1, gsem2, ssem0, ssem1, ssem2):
    XS = (xs0, xs1, xs2)
    XV = (xv0, xv1)
    PAD = (pad0, pad1)
    GSEM = (gsem0, gsem1, gsem2)
    SSEM = (ssem0, ssem1, ssem2)

    c = lax.axis_index("c")
    s = lax.axis_index("s")
    wid = s * NC + c

    zero16 = jnp.zeros((16,), jnp.float32)
    e1 = jnp.where(lax.iota(jnp.int32, 16) == 0, 1.0, 0.0).astype(jnp.float32)
    trash16 = jnp.full((16,), G, jnp.int32)

    def _fill_z(k, carry):
        r = k // (DS // 16)
        cc = k % (DS // 16)
        pad0[r, pl.ds(cc * 16, 16)] = zero16
        return carry
    lax.fori_loop(0, ZR * (DS // 16), _fill_z, 0)
    row0 = s * ZR
    pltpu.sync_copy(pad0.at[pl.ds(0, ZR)], shared_s.at[pl.ds(row0, ZR)])
    pltpu.sync_copy(pad0.at[pl.ds(0, ZR)], shared_v.at[pl.ds(row0, ZR)])

    for p in range(2):
        pad_p = PAD[p]

        def _fill_ctl(k, carry, pad_p=pad_p):
            r = k // 5
            m = k % 5
            val = jnp.where(m == 0, e1, zero16)
            pad_p[r, pl.ds((3 + m) * 16, 16)] = val
            return carry
        lax.fori_loop(0, C * 5, _fill_ctl, 0)

    plsc.subcore_barrier()

    def batch_of(t):
        b = t * NW + wid
        b_eff = jnp.minimum(b, NB_FULL - 1)
        base = pl.multiple_of(b_eff * C, C)
        return b, base

    handles = {}

    def issue_gather(t):
        p = t % NBUF
        _, base = batch_of(t)
        handles[(t, 'xs')] = pltpu.async_copy(
            xs_hbm.at[pl.ds(base, C)], XS[p], GSEM[p])
        handles[(t, 'xv')] = pltpu.async_copy(
            xv_hbm.at[pl.ds((base // DS) * DV, VR)], XV[t % 2], GSEM[p])
        handles[(t, 'ix')] = pltpu.async_copy(
            i_hbm.at[pl.ds(base, C)], idx2.at[p], GSEM[p])

    def process(t):
        p = t % NBUF
        handles.pop((t, 'xs')).wait()
        handles.pop((t, 'xv')).wait()
        handles.pop((t, 'ix')).wait()
        b, _ = batch_of(t)

        @pl.when(b >= NB_FULL)
        def _trash():
            idx_row = idx2.at[p]
            for k in range(C // 16):
                idx_row[pl.ds(k * 16, 16)] = trash16

        xv_p, pad_p = XV[t % 2], PAD[t % 2]

        def _pad_cp(r, carry):
            o = r * DV
            pad_p[r, pl.ds(0, 16)] = xv_p[o // DS, pl.ds(o % DS, 16)]
            o = o + 16
            pad_p[r, pl.ds(16, 16)] = xv_p[o // DS, pl.ds(o % DS, 16)]
            o = o + 16
            pad_p[r, pl.ds(32, 16)] = xv_p[o // DS, pl.ds(o % DS, 16)]
            return carry
        lax.fori_loop(0, C, _pad_cp, 0)

        handles[(t, 'ss')] = pltpu.async_copy(
            XS[p], shared_s.at[idx2.at[p]], SSEM[p], add=True)
        handles[(t, 'sv')] = pltpu.async_copy(
            pad_p, shared_v.at[idx2.at[p]], SSEM[p], add=True)

    def wait_scatter(t):
        handles.pop((t, 'ss')).wait()
        handles.pop((t, 'sv')).wait()

    issue_gather(0)
    issue_gather(1)
    for t in range(T_STEPS):
        if t - NBUF + 1 >= 0:
            wait_scatter(t - NBUF + 1)
        if t + 1 < T_STEPS and t + 1 >= 2:
            issue_gather(t + 1)
        process(t)
    for t in range(max(0, T_STEPS - NBUF + 1), T_STEPS):
        wait_scatter(t)

    @pl.when(wid == NW - 1)
    def _tail():
        base = NB_FULL * C
        pltpu.sync_copy(xs_hbm.at[pl.ds(base, TAIL)], xs0.at[pl.ds(0, TAIL)])
        pltpu.sync_copy(xv_hbm.at[pl.ds((base // DS) * DV, TAIL * DV // DS)],
                        xv0.at[pl.ds(0, TAIL * DV // DS)])
        pltpu.sync_copy(i_hbm.at[pl.ds(base, TAIL)], idxt)

        def _pad_cp_t(r, carry):
            o = r * DV
            pad0[r, pl.ds(0, 16)] = xv0[o // DS, pl.ds(o % DS, 16)]
            o = o + 16
            pad0[r, pl.ds(16, 16)] = xv0[o // DS, pl.ds(o % DS, 16)]
            o = o + 16
            pad0[r, pl.ds(32, 16)] = xv0[o // DS, pl.ds(o % DS, 16)]
            return carry
        lax.fori_loop(0, TAIL, _pad_cp_t, 0)
        pltpu.sync_copy(xs0.at[pl.ds(0, TAIL)], shared_s.at[idxt], add=True)
        pltpu.sync_copy(pad0.at[pl.ds(0, TAIL)], shared_v.at[idxt], add=True)

    plsc.subcore_barrier()

    @pl.when(s == 0)
    def _out():
        pltpu.sync_copy(shared_s, ps_hbm.at[c])
        pltpu.sync_copy(shared_v, pv_hbm.at[c])


@functools.lru_cache(maxsize=1)
def _build_segsum():
    mesh = plsc.VectorSubcoreMesh(core_axis_name="c", subcore_axis_name="s")
    return pl.kernel(
        _segsum_body,
        mesh=mesh,
        out_type=[
            jax.ShapeDtypeStruct((NC, GP, DS), jnp.float32),
            jax.ShapeDtypeStruct((NC, GP, DS), jnp.float32),
        ],
        scratch_types=(
            [pltpu.VMEM((C, DS), jnp.float32)] * NBUF
            + [pltpu.VMEM((VR, DS), jnp.float32)] * 2
            + [pltpu.VMEM((C, DS), jnp.float32)] * 2
            + [pltpu.VMEM((NBUF, C), jnp.int32),
               pltpu.VMEM((TAIL,), jnp.int32),
               pltpu.VMEM_SHARED((GP, DS), jnp.float32),
               pltpu.VMEM_SHARED((GP, DS), jnp.float32)]
            + [pltpu.SemaphoreType.DMA] * (2 * NBUF)
        ),
    )


def _sigmoid(x):
    return 1.0 / (1.0 + jnp.exp(-x))


def _epilogue(ps, pv, us, uvf,
              wda, wdb, bd, wh1a, wh1b, wvo1, wso1a, wso1b, bso1, wg1, bg1,
              wh2, wvo2, wso2a, wso2b, bso2, wg2, bg2,
              s2o, v2o):
    dot = functools.partial(jnp.dot, preferred_element_type=jnp.float32)
    ssum = ps[0]
    vfull = pv[0]
    for k in range(1, NC):
        ssum = ssum + ps[k]
        vfull = vfull + pv[k]
    ssum = ssum[:G]
    vsum = vfull[:G, :DV]
    cnt = vfull[:G, DV:DV + 1]
    inv = 1.0 / jnp.maximum(cnt, 1.0)
    avg_s = ssum * inv
    av = vsum * inv
    s1 = dot(avg_s, wda[...]) + dot(us[...], wdb[...]) + bd[...]
    uvc = uvf[...]
    vh = []
    for d in range(3):
        avd = av[:, VI * d:VI * (d + 1)]
        vh.append(dot(avd, wh1a[...]) + uvc[:, d:d + 1] * wh1b[...])
    sh = jnp.sqrt(vh[0] * vh[0] + vh[1] * vh[1] + vh[2] * vh[2])
    so = dot(sh, wso1a[...]) + dot(s1, wso1b[...]) + bso1[...]
    g = dot(_sigmoid(so), wg1[...]) + bg1[...]
    vo = [dot(vh[d], wvo1[...]) * g for d in range(3)]
    vh2 = [dot(vo[d], wh2[...]) for d in range(3)]
    sh2 = jnp.sqrt(vh2[0] * vh2[0] + vh2[1] * vh2[1] + vh2[2] * vh2[2])
    s2 = dot(sh2, wso2a[...]) + dot(so, wso2b[...]) + bso2[...]
    g2 = dot(_sigmoid(s2), wg2[...]) + bg2[...]
    v2 = [dot(vh2[d], wvo2[...]) * g2 for d in range(3)]
    s2o[...] = s2
    v2o[...] = jnp.concatenate(v2, axis=1)


def kernel(x_s, x_v, i, u_s, u_v, W_dense, b_dense,
           Wh1, Wvo1, Wso1, bso1, Wg1, bg1,
           Wh2, Wvo2, Wso2, bso2, Wg2, bg2):
    xv = x_v.reshape(N * DV // DS, DS)
    ps, pv = _build_segsum()(x_s, xv, i.astype(jnp.int32))
    uvf = u_v.reshape(G, 3)
    args = (
        ps, pv, u_s, uvf,
        W_dense[:DS], W_dense[DS:], b_dense.reshape(1, 16),
        Wh1[:VI], Wh1[VI:VI + 1], Wvo1,
        Wso1[:17], Wso1[17:], bso1.reshape(1, 8), Wg1, bg1.reshape(1, 3),
        Wh2, Wvo2,
        Wso2[:3], Wso2[3:], bso2.reshape(1, 3), Wg2, bg2.reshape(1, 3),
    )
    s2, v2 = pl.pallas_call(
        _epilogue,
        out_shape=[
            jax.ShapeDtypeStruct((G, 3), jnp.float32),
            jax.ShapeDtypeStruct((G, 9), jnp.float32),
        ],
    )(*args)
    return (s2, v2.reshape(G, 3, 3))


# speedup vs baseline: 3.5714x; 3.5714x over previous
import functools

import jax
import jax.numpy as jnp
from jax import lax
from jax.experimental import pallas as pl
from jax.experimental.pallas import tpu as pltpu
from jax.experimental.pallas import tpu_sc as plsc

N = 100000
G = 512
DS = 128
VI = 16
DV = 3 * VI

NC, NS = 2, 16
NW = NC * NS

C = 128
NB_FULL = N // C
TAIL = N - NB_FULL * C
T_STEPS = -(-NB_FULL // NW)
ZR = G // NS
GP = G + 8
NBUF = 3

BV = 2048
NPAD = -(-N // BV) * BV
KB = NPAD // BV


def _segsum_body(xs_hbm, i_hbm, ps_hbm,
                 xs0, xs1, xs2, idx2, idxt, shared_s,
                 gsem0, gsem1, gsem2, ssem0, ssem1, ssem2):
    XS = (xs0, xs1, xs2)
    GSEM = (gsem0, gsem1, gsem2)
    SSEM = (ssem0, ssem1, ssem2)

    c = lax.axis_index("c")
    s = lax.axis_index("s")
    wid = s * NC + c

    zero16 = jnp.zeros((16,), jnp.float32)
    trash16 = jnp.full((16,), G, jnp.int32)

    def _fill_z(k, carry):
        r = k // (DS // 16)
        cc = k % (DS // 16)
        xs0[r, pl.ds(cc * 16, 16)] = zero16
        return carry
    lax.fori_loop(0, ZR * (DS // 16), _fill_z, 0)
    pltpu.sync_copy(xs0.at[pl.ds(0, ZR)], shared_s.at[pl.ds(s * ZR, ZR)])
    plsc.subcore_barrier()

    def batch_of(t):
        b = t * NW + wid
        b_eff = jnp.minimum(b, NB_FULL - 1)
        base = pl.multiple_of(b_eff * C, C)
        return b, base

    handles = {}

    def issue_gather(t):
        p = t % NBUF
        _, base = batch_of(t)
        handles[(t, 'xs')] = pltpu.async_copy(
            xs_hbm.at[pl.ds(base, C)], XS[p], GSEM[p])
        handles[(t, 'ix')] = pltpu.async_copy(
            i_hbm.at[pl.ds(base, C)], idx2.at[p], GSEM[p])

    def process(t):
        p = t % NBUF
        handles.pop((t, 'xs')).wait()
        handles.pop((t, 'ix')).wait()
        b, _ = batch_of(t)

        @pl.when(b >= NB_FULL)
        def _trash():
            idx_row = idx2.at[p]
            for k in range(C // 16):
                idx_row[pl.ds(k * 16, 16)] = trash16

        handles[(t, 'ss')] = pltpu.async_copy(
            XS[p], shared_s.at[idx2.at[p]], SSEM[p], add=True)

    def wait_scatter(t):
        handles.pop((t, 'ss')).wait()

    issue_gather(0)
    issue_gather(1)
    for t in range(T_STEPS):
        if t - NBUF + 1 >= 0:
            wait_scatter(t - NBUF + 1)
        if t + 1 < T_STEPS and t + 1 >= 2:
            issue_gather(t + 1)
        process(t)
    for t in range(max(0, T_STEPS - NBUF + 1), T_STEPS):
        wait_scatter(t)

    @pl.when(wid == NW - 1)
    def _tail():
        base = NB_FULL * C
        pltpu.sync_copy(xs_hbm.at[pl.ds(base, TAIL)], xs0.at[pl.ds(0, TAIL)])
        pltpu.sync_copy(i_hbm.at[pl.ds(base, TAIL)], idxt)
        pltpu.sync_copy(xs0.at[pl.ds(0, TAIL)], shared_s.at[idxt], add=True)

    plsc.subcore_barrier()

    @pl.when(s == 0)
    def _out():
        pltpu.sync_copy(shared_s, ps_hbm.at[c])


@functools.lru_cache(maxsize=1)
def _build_segsum():
    mesh = plsc.VectorSubcoreMesh(core_axis_name="c", subcore_axis_name="s")
    return pl.kernel(
        _segsum_body,
        mesh=mesh,
        out_type=[
            jax.ShapeDtypeStruct((NC, GP, DS), jnp.float32),
        ],
        scratch_types=(
            [pltpu.VMEM((C, DS), jnp.float32)] * NBUF
            + [pltpu.VMEM((NBUF, C), jnp.int32),
               pltpu.VMEM((TAIL,), jnp.int32),
               pltpu.VMEM_SHARED((GP, DS), jnp.float32)]
            + [pltpu.SemaphoreType.DMA] * (2 * NBUF)
        ),
    )


def _vsum_body(ids_ref, xv_ref, vout_ref, cout_ref):
    k = pl.program_id(0)
    ids = ids_ref[0, 0, :].reshape(1, BV)
    iota = lax.broadcasted_iota(jnp.int32, (G, BV), 0)
    oh = (iota == ids).astype(jnp.float32)
    xvb = jnp.where(ids < G, xv_ref[...], 0.0)
    pv = lax.dot_general(oh, xvb, (((1,), (1,)), ((), ())),
                         preferred_element_type=jnp.float32)
    pc = jnp.sum(oh, axis=1, keepdims=True)

    @pl.when(k == 0)
    def _init():
        vout_ref[...] = pv
        cout_ref[...] = pc

    @pl.when(k != 0)
    def _acc():
        vout_ref[...] += pv
        cout_ref[...] += pc


def _sigmoid(x):
    return 1.0 / (1.0 + jnp.exp(-x))


def _epilogue(ps, vs, cs, us, uvf,
              wda, wdb, bd, wh1a, wh1b, wvo1, wso1a, wso1b, bso1, wg1, bg1,
              wh2, wvo2, wso2a, wso2b, bso2, wg2, bg2,
              s2o, v2o):
    dot = functools.partial(jnp.dot, preferred_element_type=jnp.float32)
    ssum = ps[0]
    for k in range(1, NC):
        ssum = ssum + ps[k]
    ssum = ssum[:G]
    vsum = vs[...]
    cnt = cs[...]
    inv = 1.0 / jnp.maximum(cnt, 1.0)
    avg_s = ssum * inv
    av = vsum * inv
    s1 = dot(avg_s, wda[...]) + dot(us[...], wdb[...]) + bd[...]
    uvc = uvf[...]
    vh = []
    for d in range(3):
        avd = av[:, VI * d:VI * (d + 1)]
        vh.append(dot(avd, wh1a[...]) + uvc[:, d:d + 1] * wh1b[...])
    sh = jnp.sqrt(vh[0] * vh[0] + vh[1] * vh[1] + vh[2] * vh[2])
    so = dot(sh, wso1a[...]) + dot(s1, wso1b[...]) + bso1[...]
    g = dot(_sigmoid(so), wg1[...]) + bg1[...]
    vo = [dot(vh[d], wvo1[...]) * g for d in range(3)]
    vh2 = [dot(vo[d], wh2[...]) for d in range(3)]
    sh2 = jnp.sqrt(vh2[0] * vh2[0] + vh2[1] * vh2[1] + vh2[2] * vh2[2])
    s2 = dot(sh2, wso2a[...]) + dot(so, wso2b[...]) + bso2[...]
    g2 = dot(_sigmoid(s2), wg2[...]) + bg2[...]
    v2 = [dot(vh2[d], wvo2[...]) * g2 for d in range(3)]
    s2o[...] = s2
    v2o[...] = jnp.concatenate(v2, axis=1)


def kernel(x_s, x_v, i, u_s, u_v, W_dense, b_dense,
           Wh1, Wvo1, Wso1, bso1, Wg1, bg1,
           Wh2, Wvo2, Wso2, bso2, Wg2, bg2):
    ii = i.astype(jnp.int32)
    (ps,) = _build_segsum()(x_s, ii)

    xvt = jnp.transpose(x_v, (1, 2, 0)).reshape(DV, N)
    ids_pad = jnp.concatenate(
        [ii, jnp.full((NPAD - N,), G, jnp.int32)]).reshape(KB, 1, BV)
    vs, cs = pl.pallas_call(
        _vsum_body,
        grid=(KB,),
        in_specs=[
            pl.BlockSpec((1, 1, BV), lambda k: (k, 0, 0)),
            pl.BlockSpec((DV, BV), lambda k: (0, k)),
        ],
        out_specs=[
            pl.BlockSpec((G, DV), lambda k: (0, 0)),
            pl.BlockSpec((G, 1), lambda k: (0, 0)),
        ],
        out_shape=[
            jax.ShapeDtypeStruct((G, DV), jnp.float32),
            jax.ShapeDtypeStruct((G, 1), jnp.float32),
        ],
    )(ids_pad, xvt)

    uvf = u_v.reshape(G, 3)
    args = (
        ps, vs, cs, u_s, uvf,
        W_dense[:DS], W_dense[DS:], b_dense.reshape(1, 16),
        Wh1[:VI], Wh1[VI:VI + 1], Wvo1,
        Wso1[:17], Wso1[17:], bso1.reshape(1, 8), Wg1, bg1.reshape(1, 3),
        Wh2, Wvo2,
        Wso2[:3], Wso2[3:], bso2.reshape(1, 3), Wg2, bg2.reshape(1, 3),
    )
    s2, v2 = pl.pallas_call(
        _epilogue,
        out_shape=[
            jax.ShapeDtypeStruct((G, 3), jnp.float32),
            jax.ShapeDtypeStruct((G, 9), jnp.float32),
        ],
    )(*args)
    return (s2, v2.reshape(G, 3, 3))
